# baseline (speedup 1.0000x reference)
import functools

import jax
import jax.numpy as jnp
from jax import lax
from jax.experimental import pallas as pl
from jax.experimental.pallas import tpu as pltpu
from jax.experimental.pallas import tpu_sc as plsc

EMB = 16
BATCH = 16384
TBL = 1000000
NC = 2
NS = 16
NW = NC * NS
BPW = BATCH // NW
CH = 128
L = 16


def _gather_body(u_hbm, i_hbm, ut_hbm, it_hbm, ue_out, ie_out,
                 idx_u, idx_i, idxf, rows_u, rows_i, sem):
    wid = lax.axis_index("s") * NC + lax.axis_index("c")
    base = wid * BPW
    pltpu.sync_copy(u_hbm.at[pl.ds(base, BPW)], idx_u)
    pltpu.sync_copy(i_hbm.at[pl.ds(base, BPW)], idx_i)
    for src_idx, table, rows in ((idx_u, ut_hbm, rows_u),
                                 (idx_i, it_hbm, rows_i)):
        for g in range(BPW // L):
            v = src_idx[pl.ds(g * L, L)]
            for c in range(EMB):
                idxf[c, pl.ds(g * L, L)] = v + (c * TBL)
        copies = []
        for c in range(EMB):
            for ch in range(BPW // CH):
                off = ch * CH
                copies.append(pltpu.async_copy(
                    table.at[idxf.at[c, pl.ds(off, CH)]],
                    rows.at[c, pl.ds(off, CH)], sem))
        for cp in copies:
            cp.wait()
    pltpu.sync_copy(rows_u, ue_out.at[:, pl.ds(base, BPW)])
    pltpu.sync_copy(rows_i, ie_out.at[:, pl.ds(base, BPW)])


def _sc_gather(u, i, ut_flat, it_flat):
    mesh = plsc.VectorSubcoreMesh(core_axis_name="c", subcore_axis_name="s")
    f = functools.partial(
        pl.kernel,
        mesh=mesh,
        out_type=[
            jax.ShapeDtypeStruct((EMB, BATCH), jnp.float32),
            jax.ShapeDtypeStruct((EMB, BATCH), jnp.float32),
        ],
        scratch_types=[
            pltpu.VMEM((BPW,), jnp.int32),
            pltpu.VMEM((BPW,), jnp.int32),
            pltpu.VMEM((EMB, BPW), jnp.int32),
            pltpu.VMEM((EMB, BPW), jnp.float32),
            pltpu.VMEM((EMB, BPW), jnp.float32),
            pltpu.SemaphoreType.DMA,
        ],
        compiler_params=pltpu.CompilerParams(use_tc_tiling_on_sc=False),
    )(_gather_body)
    return f(u, i, ut_flat, it_flat)


def _mlp_body(ue_ref, ie_ref, w1a_ref, w1b_ref, b1_ref, w2_ref, b2_ref, out_ref):
    h = jnp.dot(w1a_ref[...], ue_ref[...], preferred_element_type=jnp.float32)
    h = h + jnp.dot(w1b_ref[...], ie_ref[...], preferred_element_type=jnp.float32)
    h = jnp.maximum(h + b1_ref[...], 0.0)
    o = jnp.sum(h * w2_ref[...], axis=0, keepdims=True) + b2_ref[...]
    out_ref[...] = 1.0 / (1.0 + jnp.exp(-o))


BN = 4096


def _tc_mlp(ue_t, ie_t, w1a_t, w1b_t, b1c, w2c, b2c):
    grid = (BATCH // BN,)
    return pl.pallas_call(
        _mlp_body,
        grid=grid,
        in_specs=[
            pl.BlockSpec((EMB, BN), lambda m: (0, m)),
            pl.BlockSpec((EMB, BN), lambda m: (0, m)),
            pl.BlockSpec((EMB, EMB), lambda m: (0, 0)),
            pl.BlockSpec((EMB, EMB), lambda m: (0, 0)),
            pl.BlockSpec((EMB, 1), lambda m: (0, 0)),
            pl.BlockSpec((EMB, 1), lambda m: (0, 0)),
            pl.BlockSpec((1, 1), lambda m: (0, 0)),
        ],
        out_specs=pl.BlockSpec((1, BN), lambda m: (0, m)),
        out_shape=jax.ShapeDtypeStruct((1, BATCH), jnp.float32),
    )(ue_t, ie_t, w1a_t, w1b_t, b1c, w2c, b2c)


def kernel(u, i, user_emb, item_emb, W1, b1, W2, b2):
    u = u.astype(jnp.int32)
    i = i.astype(jnp.int32)
    ut_flat = user_emb.T.reshape(-1)
    it_flat = item_emb.T.reshape(-1)
    ue_t, ie_t = _sc_gather(u, i, ut_flat, it_flat)
    w1a_t = W1[:EMB].T
    w1b_t = W1[EMB:].T
    b1c = b1.reshape(EMB, 1)
    w2c = W2.reshape(EMB, 1)
    b2c = b2.reshape(1, 1)
    out_t = _tc_mlp(ue_t, ie_t, w1a_t, w1b_t, b1c, w2c, b2c)
    return out_t.reshape(BATCH, 1)


# speedup vs baseline: 2.3383x; 2.3383x over previous
import functools

import jax
import jax.numpy as jnp
from jax import lax
from jax.experimental import pallas as pl
from jax.experimental.pallas import tpu as pltpu
from jax.experimental.pallas import tpu_sc as plsc

EMB = 16
BATCH = 16384
TBL = 1000000
NC = 2
NS = 16
NW = NC * NS
BPW = BATCH // NW
CH = 128

TCOL = 4096


def _transpose_body(ut_ref, it_ref, u_out, i_out):
    u_out[...] = ut_ref[...].T
    i_out[...] = it_ref[...].T


def _tc_transpose(ut, it):
    grid = ((TBL + TCOL - 1) // TCOL,)
    return pl.pallas_call(
        _transpose_body,
        grid=grid,
        in_specs=[
            pl.BlockSpec((EMB, TCOL), lambda m: (0, m)),
            pl.BlockSpec((EMB, TCOL), lambda m: (0, m)),
        ],
        out_specs=[
            pl.BlockSpec((TCOL, EMB), lambda m: (m, 0)),
            pl.BlockSpec((TCOL, EMB), lambda m: (m, 0)),
        ],
        out_shape=[
            jax.ShapeDtypeStruct((TBL, EMB), jnp.float32),
            jax.ShapeDtypeStruct((TBL, EMB), jnp.float32),
        ],
    )(ut, it)


def _gather_body(u_hbm, i_hbm, uemb_hbm, iemb_hbm, ue_out, ie_out,
                 idx_u, idx_i, rows_u, rows_i, sem):
    wid = lax.axis_index("s") * NC + lax.axis_index("c")
    base = wid * BPW
    pltpu.sync_copy(u_hbm.at[pl.ds(base, BPW)], idx_u)
    pltpu.sync_copy(i_hbm.at[pl.ds(base, BPW)], idx_i)
    copies = []
    for c in range(BPW // CH):
        off = c * CH
        copies.append(pltpu.async_copy(
            uemb_hbm.at[idx_u.at[pl.ds(off, CH)]],
            rows_u.at[pl.ds(off, CH), :], sem))
        copies.append(pltpu.async_copy(
            iemb_hbm.at[idx_i.at[pl.ds(off, CH)]],
            rows_i.at[pl.ds(off, CH), :], sem))
    for cp in copies:
        cp.wait()
    pltpu.sync_copy(rows_u, ue_out.at[pl.ds(base, BPW)])
    pltpu.sync_copy(rows_i, ie_out.at[pl.ds(base, BPW)])


def _sc_gather(u, i, user_emb, item_emb):
    mesh = plsc.VectorSubcoreMesh(core_axis_name="c", subcore_axis_name="s")
    f = functools.partial(
        pl.kernel,
        mesh=mesh,
        out_type=[
            jax.ShapeDtypeStruct((BATCH, EMB), jnp.float32),
            jax.ShapeDtypeStruct((BATCH, EMB), jnp.float32),
        ],
        scratch_types=[
            pltpu.VMEM((BPW,), jnp.int32),
            pltpu.VMEM((BPW,), jnp.int32),
            pltpu.VMEM((BPW, EMB), jnp.float32),
            pltpu.VMEM((BPW, EMB), jnp.float32),
            pltpu.SemaphoreType.DMA,
        ],
        compiler_params=pltpu.CompilerParams(use_tc_tiling_on_sc=False),
    )(_gather_body)
    return f(u, i, user_emb, item_emb)


def _mlp_body(ue_ref, ie_ref, w1a_ref, w1b_ref, b1_ref, w2_ref, b2_ref, out_ref):
    h = jnp.dot(ue_ref[...], w1a_ref[...], preferred_element_type=jnp.float32)
    h = h + jnp.dot(ie_ref[...], w1b_ref[...], preferred_element_type=jnp.float32)
    h = jnp.maximum(h + b1_ref[...], 0.0)
    o = jnp.sum(h * w2_ref[...], axis=1, keepdims=True) + b2_ref[...]
    out_ref[...] = 1.0 / (1.0 + jnp.exp(-o))


BM = 2048


def _tc_mlp(ue, ie, w1a, w1b, b1r, w2r, b2r):
    grid = (BATCH // BM,)
    return pl.pallas_call(
        _mlp_body,
        grid=grid,
        in_specs=[
            pl.BlockSpec((BM, EMB), lambda m: (m, 0)),
            pl.BlockSpec((BM, EMB), lambda m: (m, 0)),
            pl.BlockSpec((EMB, EMB), lambda m: (0, 0)),
            pl.BlockSpec((EMB, EMB), lambda m: (0, 0)),
            pl.BlockSpec((1, EMB), lambda m: (0, 0)),
            pl.BlockSpec((1, EMB), lambda m: (0, 0)),
            pl.BlockSpec((1, 1), lambda m: (0, 0)),
        ],
        out_specs=pl.BlockSpec((BM, 1), lambda m: (m, 0)),
        out_shape=jax.ShapeDtypeStruct((BATCH, 1), jnp.float32),
    )(ue, ie, w1a, w1b, b1r, w2r, b2r)


def kernel(u, i, user_emb, item_emb, W1, b1, W2, b2):
    u = u.astype(jnp.int32)
    i = i.astype(jnp.int32)
    u_rm, i_rm = _tc_transpose(user_emb.T, item_emb.T)
    ue, ie = _sc_gather(u, i, u_rm, i_rm)
    w1a = W1[:EMB]
    w1b = W1[EMB:]
    b1r = b1.reshape(1, EMB)
    w2r = W2.reshape(1, EMB)
    b2r = b2.reshape(1, 1)
    return _tc_mlp(ue, ie, w1a, w1b, b1r, w2r, b2r)


# speedup vs baseline: 7.2988x; 3.1215x over previous
import functools

import jax
import jax.numpy as jnp
from jax import lax
from jax.experimental import pallas as pl
from jax.experimental.pallas import tpu as pltpu
from jax.experimental.pallas import tpu_sc as plsc

EMB = 16
BATCH = 16384
TBL = 1000000
NC = 2
NS = 16
NW = NC * NS
BPW = BATCH // NW
CH = 128
L = 16
NG = BPW // L

TCOL = 4096
NBLK = (TBL + TCOL - 1) // TCOL
BLKW = 8 * TCOL
PLANE = NBLK * BLKW


def _repack_body(ut_ref, it_ref, u_out, i_out):
    def tilestream(x):
        return x.reshape(8, TCOL // 128, 128).transpose(1, 0, 2).reshape(BLKW)
    u_out[...] = tilestream(ut_ref[...])
    i_out[...] = tilestream(it_ref[...])


def _tc_repack(ut, it):
    grid = (2, NBLK)
    return pl.pallas_call(
        _repack_body,
        grid=grid,
        in_specs=[
            pl.BlockSpec((8, TCOL), lambda p, m: (p, m)),
            pl.BlockSpec((8, TCOL), lambda p, m: (p, m)),
        ],
        out_specs=[
            pl.BlockSpec((BLKW,), lambda p, m: (p * NBLK + m,)),
            pl.BlockSpec((BLKW,), lambda p, m: (p * NBLK + m,)),
        ],
        out_shape=[
            jax.ShapeDtypeStruct((2 * PLANE,), jnp.float32),
            jax.ShapeDtypeStruct((2 * PLANE,), jnp.float32),
        ],
    )(ut, it)


def _gather_body(u_hbm, i_hbm, uf_hbm, if_hbm, ue_out, ie_out,
                 idx_u, idx_i, idxf, rows_u, rows_i, sem):
    wid = lax.axis_index("s") * NC + lax.axis_index("c")
    base = wid * BPW
    pltpu.sync_copy(u_hbm.at[pl.ds(base, BPW)], idx_u)
    pltpu.sync_copy(i_hbm.at[pl.ds(base, BPW)], idx_i)
    for src_idx, table, rows in ((idx_u, uf_hbm, rows_u),
                                 (idx_i, if_hbm, rows_i)):
        for g in range(NG):
            r = src_idx[pl.ds(g * L, L)]
            t = ((r >> 7) << 10) + (r & 127)
            for c in range(EMB):
                off = (c // 8) * PLANE + (c % 8) * 128
                idxf[c, pl.ds(g * L, L)] = t + off
        copies = []
        for c in range(EMB):
            for ch in range(BPW // CH):
                off = ch * CH
                copies.append(pltpu.async_copy(
                    table.at[idxf.at[c, pl.ds(off, CH)]],
                    rows.at[c, pl.ds(off, CH)], sem))
        for cp in copies:
            cp.wait()
    pltpu.sync_copy(rows_u, ue_out.at[:, pl.ds(base, BPW)])
    pltpu.sync_copy(rows_i, ie_out.at[:, pl.ds(base, BPW)])


def _sc_gather(u, i, uf, if_):
    mesh = plsc.VectorSubcoreMesh(core_axis_name="c", subcore_axis_name="s")
    f = functools.partial(
        pl.kernel,
        mesh=mesh,
        out_type=[
            jax.ShapeDtypeStruct((EMB, BATCH), jnp.float32),
            jax.ShapeDtypeStruct((EMB, BATCH), jnp.float32),
        ],
        scratch_types=[
            pltpu.VMEM((BPW,), jnp.int32),
            pltpu.VMEM((BPW,), jnp.int32),
            pltpu.VMEM((EMB, BPW), jnp.int32),
            pltpu.VMEM((EMB, BPW), jnp.float32),
            pltpu.VMEM((EMB, BPW), jnp.float32),
            pltpu.SemaphoreType.DMA,
        ],
        compiler_params=pltpu.CompilerParams(use_tc_tiling_on_sc=False),
    )(_gather_body)
    return f(u, i, uf, if_)


def _mlp_body(ue_ref, ie_ref, w1a_ref, w1b_ref, b1_ref, w2_ref, b2_ref, out_ref):
    h = jnp.dot(w1a_ref[...], ue_ref[...], preferred_element_type=jnp.float32)
    h = h + jnp.dot(w1b_ref[...], ie_ref[...], preferred_element_type=jnp.float32)
    h = jnp.maximum(h + b1_ref[...], 0.0)
    o = jnp.sum(h * w2_ref[...], axis=0, keepdims=True) + b2_ref[...]
    out_ref[...] = 1.0 / (1.0 + jnp.exp(-o))


BN = 4096


def _tc_mlp(ue_t, ie_t, w1a_t, w1b_t, b1c, w2c, b2c):
    grid = (BATCH // BN,)
    return pl.pallas_call(
        _mlp_body,
        grid=grid,
        in_specs=[
            pl.BlockSpec((EMB, BN), lambda m: (0, m)),
            pl.BlockSpec((EMB, BN), lambda m: (0, m)),
            pl.BlockSpec((EMB, EMB), lambda m: (0, 0)),
            pl.BlockSpec((EMB, EMB), lambda m: (0, 0)),
            pl.BlockSpec((EMB, 1), lambda m: (0, 0)),
            pl.BlockSpec((EMB, 1), lambda m: (0, 0)),
            pl.BlockSpec((1, 1), lambda m: (0, 0)),
        ],
        out_specs=pl.BlockSpec((1, BN), lambda m: (0, m)),
        out_shape=jax.ShapeDtypeStruct((1, BATCH), jnp.float32),
    )(ue_t, ie_t, w1a_t, w1b_t, b1c, w2c, b2c)


def kernel(u, i, user_emb, item_emb, W1, b1, W2, b2):
    u = u.astype(jnp.int32)
    i = i.astype(jnp.int32)
    uf, if_ = _tc_repack(user_emb.T, item_emb.T)
    ue_t, ie_t = _sc_gather(u, i, uf, if_)
    w1a_t = W1[:EMB].T
    w1b_t = W1[EMB:].T
    b1c = b1.reshape(EMB, 1)
    w2c = W2.reshape(EMB, 1)
    b2c = b2.reshape(1, 1)
    out_t = _tc_mlp(ue_t, ie_t, w1a_t, w1b_t, b1c, w2c, b2c)
    return out_t.reshape(BATCH, 1)


# speedup vs baseline: 13.7449x; 1.8832x over previous
import functools

import jax
import jax.numpy as jnp
from jax import lax
from jax.experimental import pallas as pl
from jax.experimental.pallas import tpu as pltpu
from jax.experimental.pallas import tpu_sc as plsc

EMB = 16
BATCH = 16384
TBL = 1000000
NC = 2
NS = 16
NW = NC * NS
BPW = BATCH // NW
CH = 128
L = 16
NG = BPW // L

TCOL = 16384
NBLK = (TBL + TCOL - 1) // TCOL
BLKW = 8 * TCOL
PLANE = NBLK * BLKW


def _repack_body(ut_ref, it_ref, u_out, i_out):
    def tilestream(x):
        return x.reshape(8, TCOL // 128, 128).transpose(1, 0, 2).reshape(BLKW)
    u_out[...] = tilestream(ut_ref[...])
    i_out[...] = tilestream(it_ref[...])


def _tc_repack(ut, it):
    grid = (2, NBLK)
    return pl.pallas_call(
        _repack_body,
        grid=grid,
        in_specs=[
            pl.BlockSpec((8, TCOL), lambda p, m: (p, m)),
            pl.BlockSpec((8, TCOL), lambda p, m: (p, m)),
        ],
        out_specs=[
            pl.BlockSpec((BLKW,), lambda p, m: (p * NBLK + m,)),
            pl.BlockSpec((BLKW,), lambda p, m: (p * NBLK + m,)),
        ],
        out_shape=[
            jax.ShapeDtypeStruct((2 * PLANE,), jnp.float32),
            jax.ShapeDtypeStruct((2 * PLANE,), jnp.float32),
        ],
    )(ut, it)


def _gather_body(u_hbm, i_hbm, uf_hbm, if_hbm, ue_out, ie_out,
                 idx_u, idx_i, idxf, rows_u, rows_i, sem):
    wid = lax.axis_index("s") * NC + lax.axis_index("c")
    base = wid * BPW
    pltpu.sync_copy(u_hbm.at[pl.ds(base, BPW)], idx_u)
    pltpu.sync_copy(i_hbm.at[pl.ds(base, BPW)], idx_i)
    for src_idx, table, rows in ((idx_u, uf_hbm, rows_u),
                                 (idx_i, if_hbm, rows_i)):
        for g in range(NG):
            r = src_idx[pl.ds(g * L, L)]
            t = ((r >> 7) << 10) + (r & 127)
            for c in range(EMB):
                off = (c // 8) * PLANE + (c % 8) * 128
                idxf[c, pl.ds(g * L, L)] = t + off
        copies = []
        for c in range(EMB):
            for ch in range(BPW // CH):
                off = ch * CH
                copies.append(pltpu.async_copy(
                    table.at[idxf.at[c, pl.ds(off, CH)]],
                    rows.at[c, pl.ds(off, CH)], sem))
        for cp in copies:
            cp.wait()
    pltpu.sync_copy(rows_u, ue_out.at[:, pl.ds(base, BPW)])
    pltpu.sync_copy(rows_i, ie_out.at[:, pl.ds(base, BPW)])


def _sc_gather(u, i, uf, if_):
    mesh = plsc.VectorSubcoreMesh(core_axis_name="c", subcore_axis_name="s")
    f = functools.partial(
        pl.kernel,
        mesh=mesh,
        out_type=[
            jax.ShapeDtypeStruct((EMB, BATCH), jnp.float32),
            jax.ShapeDtypeStruct((EMB, BATCH), jnp.float32),
        ],
        scratch_types=[
            pltpu.VMEM((BPW,), jnp.int32),
            pltpu.VMEM((BPW,), jnp.int32),
            pltpu.VMEM((EMB, BPW), jnp.int32),
            pltpu.VMEM((EMB, BPW), jnp.float32),
            pltpu.VMEM((EMB, BPW), jnp.float32),
            pltpu.SemaphoreType.DMA,
        ],
        compiler_params=pltpu.CompilerParams(use_tc_tiling_on_sc=False),
    )(_gather_body)
    return f(u, i, uf, if_)


def _mlp_body(ue_ref, ie_ref, w1a_ref, w1b_ref, b1_ref, w2_ref, b2_ref, out_ref):
    h = jnp.dot(w1a_ref[...], ue_ref[...], preferred_element_type=jnp.float32)
    h = h + jnp.dot(w1b_ref[...], ie_ref[...], preferred_element_type=jnp.float32)
    h = jnp.maximum(h + b1_ref[...], 0.0)
    o = jnp.sum(h * w2_ref[...], axis=0, keepdims=True) + b2_ref[...]
    out_ref[...] = 1.0 / (1.0 + jnp.exp(-o))


BN = 4096


def _tc_mlp(ue_t, ie_t, w1a_t, w1b_t, b1c, w2c, b2c):
    grid = (BATCH // BN,)
    return pl.pallas_call(
        _mlp_body,
        grid=grid,
        in_specs=[
            pl.BlockSpec((EMB, BN), lambda m: (0, m)),
            pl.BlockSpec((EMB, BN), lambda m: (0, m)),
            pl.BlockSpec((EMB, EMB), lambda m: (0, 0)),
            pl.BlockSpec((EMB, EMB), lambda m: (0, 0)),
            pl.BlockSpec((EMB, 1), lambda m: (0, 0)),
            pl.BlockSpec((EMB, 1), lambda m: (0, 0)),
            pl.BlockSpec((1, 1), lambda m: (0, 0)),
        ],
        out_specs=pl.BlockSpec((1, BN), lambda m: (0, m)),
        out_shape=jax.ShapeDtypeStruct((1, BATCH), jnp.float32),
    )(ue_t, ie_t, w1a_t, w1b_t, b1c, w2c, b2c)


def kernel(u, i, user_emb, item_emb, W1, b1, W2, b2):
    u = u.astype(jnp.int32)
    i = i.astype(jnp.int32)
    uf, if_ = _tc_repack(user_emb.T, item_emb.T)
    ue_t, ie_t = _sc_gather(u, i, uf, if_)
    w1a_t = W1[:EMB].T
    w1b_t = W1[EMB:].T
    b1c = b1.reshape(EMB, 1)
    w2c = W2.reshape(EMB, 1)
    b2c = b2.reshape(1, 1)
    out_t = _tc_mlp(ue_t, ie_t, w1a_t, w1b_t, b1c, w2c, b2c)
    return out_t.reshape(BATCH, 1)


# speedup vs baseline: 19.1098x; 1.3903x over previous
import functools

import jax
import jax.numpy as jnp
from jax import lax
from jax.experimental import pallas as pl
from jax.experimental.pallas import tpu as pltpu
from jax.experimental.pallas import tpu_sc as plsc

EMB = 16
BATCH = 16384
TBL = 1000000
NC = 2
NS = 16
NW = NC * NS
BPW = BATCH // NW
CH = 128
L = 16
NG = BPW // L

TCOL = 65536
NBLK = (TBL + TCOL - 1) // TCOL
BLKW = 8 * TCOL
PLANE = NBLK * BLKW


def _repack_body(ut_ref, it_ref, u_out, i_out):
    def tilestream(x):
        return x.reshape(8, TCOL // 128, 128).transpose(1, 0, 2).reshape(BLKW)
    u_out[...] = tilestream(ut_ref[...])
    i_out[...] = tilestream(it_ref[...])


def _tc_repack(ut, it):
    grid = (2, NBLK)
    return pl.pallas_call(
        _repack_body,
        grid=grid,
        in_specs=[
            pl.BlockSpec((8, TCOL), lambda p, m: (p, m)),
            pl.BlockSpec((8, TCOL), lambda p, m: (p, m)),
        ],
        out_specs=[
            pl.BlockSpec((BLKW,), lambda p, m: (p * NBLK + m,)),
            pl.BlockSpec((BLKW,), lambda p, m: (p * NBLK + m,)),
        ],
        out_shape=[
            jax.ShapeDtypeStruct((2 * PLANE,), jnp.float32),
            jax.ShapeDtypeStruct((2 * PLANE,), jnp.float32),
        ],
    )(ut, it)


def _gather_body(u_hbm, i_hbm, uf_hbm, if_hbm, ue_out, ie_out,
                 idx_u, idx_i, idxf, rows_u, rows_i, sem):
    wid = lax.axis_index("s") * NC + lax.axis_index("c")
    base = wid * BPW
    pltpu.sync_copy(u_hbm.at[pl.ds(base, BPW)], idx_u)
    pltpu.sync_copy(i_hbm.at[pl.ds(base, BPW)], idx_i)
    for src_idx, table, rows in ((idx_u, uf_hbm, rows_u),
                                 (idx_i, if_hbm, rows_i)):
        for g in range(NG):
            r = src_idx[pl.ds(g * L, L)]
            t = ((r >> 7) << 10) + (r & 127)
            for c in range(EMB):
                off = (c // 8) * PLANE + (c % 8) * 128
                idxf[c, pl.ds(g * L, L)] = t + off
        copies = []
        for c in range(EMB):
            for ch in range(BPW // CH):
                off = ch * CH
                copies.append(pltpu.async_copy(
                    table.at[idxf.at[c, pl.ds(off, CH)]],
                    rows.at[c, pl.ds(off, CH)], sem))
        for cp in copies:
            cp.wait()
    pltpu.sync_copy(rows_u, ue_out.at[:, pl.ds(base, BPW)])
    pltpu.sync_copy(rows_i, ie_out.at[:, pl.ds(base, BPW)])


def _sc_gather(u, i, uf, if_):
    mesh = plsc.VectorSubcoreMesh(core_axis_name="c", subcore_axis_name="s")
    f = functools.partial(
        pl.kernel,
        mesh=mesh,
        out_type=[
            jax.ShapeDtypeStruct((EMB, BATCH), jnp.float32),
            jax.ShapeDtypeStruct((EMB, BATCH), jnp.float32),
        ],
        scratch_types=[
            pltpu.VMEM((BPW,), jnp.int32),
            pltpu.VMEM((BPW,), jnp.int32),
            pltpu.VMEM((EMB, BPW), jnp.int32),
            pltpu.VMEM((EMB, BPW), jnp.float32),
            pltpu.VMEM((EMB, BPW), jnp.float32),
            pltpu.SemaphoreType.DMA,
        ],
        compiler_params=pltpu.CompilerParams(use_tc_tiling_on_sc=False),
    )(_gather_body)
    return f(u, i, uf, if_)


def _mlp_body(ue_ref, ie_ref, w1a_ref, w1b_ref, b1_ref, w2_ref, b2_ref, out_ref):
    h = jnp.dot(w1a_ref[...], ue_ref[...], preferred_element_type=jnp.float32)
    h = h + jnp.dot(w1b_ref[...], ie_ref[...], preferred_element_type=jnp.float32)
    h = jnp.maximum(h + b1_ref[...], 0.0)
    o = jnp.sum(h * w2_ref[...], axis=0, keepdims=True) + b2_ref[...]
    out_ref[...] = 1.0 / (1.0 + jnp.exp(-o))


BN = 4096


def _tc_mlp(ue_t, ie_t, w1a_t, w1b_t, b1c, w2c, b2c):
    grid = (BATCH // BN,)
    return pl.pallas_call(
        _mlp_body,
        grid=grid,
        in_specs=[
            pl.BlockSpec((EMB, BN), lambda m: (0, m)),
            pl.BlockSpec((EMB, BN), lambda m: (0, m)),
            pl.BlockSpec((EMB, EMB), lambda m: (0, 0)),
            pl.BlockSpec((EMB, EMB), lambda m: (0, 0)),
            pl.BlockSpec((EMB, 1), lambda m: (0, 0)),
            pl.BlockSpec((EMB, 1), lambda m: (0, 0)),
            pl.BlockSpec((1, 1), lambda m: (0, 0)),
        ],
        out_specs=pl.BlockSpec((1, BN), lambda m: (0, m)),
        out_shape=jax.ShapeDtypeStruct((1, BATCH), jnp.float32),
    )(ue_t, ie_t, w1a_t, w1b_t, b1c, w2c, b2c)


def kernel(u, i, user_emb, item_emb, W1, b1, W2, b2):
    u = u.astype(jnp.int32)
    i = i.astype(jnp.int32)
    uf, if_ = _tc_repack(user_emb.T, item_emb.T)
    ue_t, ie_t = _sc_gather(u, i, uf, if_)
    w1a_t = W1[:EMB].T
    w1b_t = W1[EMB:].T
    b1c = b1.reshape(EMB, 1)
    w2c = W2.reshape(EMB, 1)
    b2c = b2.reshape(1, 1)
    out_t = _tc_mlp(ue_t, ie_t, w1a_t, w1b_t, b1c, w2c, b2c)
    return out_t.reshape(BATCH, 1)


# speedup vs baseline: 19.6661x; 1.0291x over previous
import functools

import jax
import jax.numpy as jnp
from jax import lax
from jax.experimental import pallas as pl
from jax.experimental.pallas import tpu as pltpu
from jax.experimental.pallas import tpu_sc as plsc

EMB = 16
BATCH = 16384
TBL = 1000000
NC = 2
NS = 16
NW = NC * NS
BPW = BATCH // NW
CH = 128
L = 16
NG = BPW // L

TCOL = 131072
NBLK = (TBL + TCOL - 1) // TCOL
BLKW = 8 * TCOL
PLANE = NBLK * BLKW


def _repack_body(ut_ref, it_ref, u_out, i_out):
    def tilestream(x):
        return x.reshape(8, TCOL // 128, 128).transpose(1, 0, 2).reshape(BLKW)
    u_out[...] = tilestream(ut_ref[...])
    i_out[...] = tilestream(it_ref[...])


def _tc_repack(ut, it):
    grid = (2, NBLK)
    return pl.pallas_call(
        _repack_body,
        grid=grid,
        in_specs=[
            pl.BlockSpec((8, TCOL), lambda p, m: (p, m)),
            pl.BlockSpec((8, TCOL), lambda p, m: (p, m)),
        ],
        out_specs=[
            pl.BlockSpec((BLKW,), lambda p, m: (p * NBLK + m,)),
            pl.BlockSpec((BLKW,), lambda p, m: (p * NBLK + m,)),
        ],
        out_shape=[
            jax.ShapeDtypeStruct((2 * PLANE,), jnp.float32),
            jax.ShapeDtypeStruct((2 * PLANE,), jnp.float32),
        ],
    )(ut, it)


def _gather_body(u_hbm, i_hbm, uf_hbm, if_hbm, ue_out, ie_out,
                 idx_u, idx_i, idxf, rows_u, rows_i, sem):
    wid = lax.axis_index("s") * NC + lax.axis_index("c")
    base = wid * BPW
    pltpu.sync_copy(u_hbm.at[pl.ds(base, BPW)], idx_u)
    pltpu.sync_copy(i_hbm.at[pl.ds(base, BPW)], idx_i)
    for src_idx, table, rows in ((idx_u, uf_hbm, rows_u),
                                 (idx_i, if_hbm, rows_i)):
        for g in range(NG):
            r = src_idx[pl.ds(g * L, L)]
            t = ((r >> 7) << 10) + (r & 127)
            for c in range(EMB):
                off = (c // 8) * PLANE + (c % 8) * 128
                idxf[c, pl.ds(g * L, L)] = t + off
        copies = []
        for c in range(EMB):
            for ch in range(BPW // CH):
                off = ch * CH
                copies.append(pltpu.async_copy(
                    table.at[idxf.at[c, pl.ds(off, CH)]],
                    rows.at[c, pl.ds(off, CH)], sem))
        for cp in copies:
            cp.wait()
    pltpu.sync_copy(rows_u, ue_out.at[:, pl.ds(base, BPW)])
    pltpu.sync_copy(rows_i, ie_out.at[:, pl.ds(base, BPW)])


def _sc_gather(u, i, uf, if_):
    mesh = plsc.VectorSubcoreMesh(core_axis_name="c", subcore_axis_name="s")
    f = functools.partial(
        pl.kernel,
        mesh=mesh,
        out_type=[
            jax.ShapeDtypeStruct((EMB, BATCH), jnp.float32),
            jax.ShapeDtypeStruct((EMB, BATCH), jnp.float32),
        ],
        scratch_types=[
            pltpu.VMEM((BPW,), jnp.int32),
            pltpu.VMEM((BPW,), jnp.int32),
            pltpu.VMEM((EMB, BPW), jnp.int32),
            pltpu.VMEM((EMB, BPW), jnp.float32),
            pltpu.VMEM((EMB, BPW), jnp.float32),
            pltpu.SemaphoreType.DMA,
        ],
        compiler_params=pltpu.CompilerParams(use_tc_tiling_on_sc=False),
    )(_gather_body)
    return f(u, i, uf, if_)


def _mlp_body(ue_ref, ie_ref, w1a_ref, w1b_ref, b1_ref, w2_ref, b2_ref, out_ref):
    h = jnp.dot(w1a_ref[...], ue_ref[...], preferred_element_type=jnp.float32)
    h = h + jnp.dot(w1b_ref[...], ie_ref[...], preferred_element_type=jnp.float32)
    h = jnp.maximum(h + b1_ref[...], 0.0)
    o = jnp.sum(h * w2_ref[...], axis=0, keepdims=True) + b2_ref[...]
    out_ref[...] = 1.0 / (1.0 + jnp.exp(-o))


BN = 4096


def _tc_mlp(ue_t, ie_t, w1a_t, w1b_t, b1c, w2c, b2c):
    grid = (BATCH // BN,)
    return pl.pallas_call(
        _mlp_body,
        grid=grid,
        in_specs=[
            pl.BlockSpec((EMB, BN), lambda m: (0, m)),
            pl.BlockSpec((EMB, BN), lambda m: (0, m)),
            pl.BlockSpec((EMB, EMB), lambda m: (0, 0)),
            pl.BlockSpec((EMB, EMB), lambda m: (0, 0)),
            pl.BlockSpec((EMB, 1), lambda m: (0, 0)),
            pl.BlockSpec((EMB, 1), lambda m: (0, 0)),
            pl.BlockSpec((1, 1), lambda m: (0, 0)),
        ],
        out_specs=pl.BlockSpec((1, BN), lambda m: (0, m)),
        out_shape=jax.ShapeDtypeStruct((1, BATCH), jnp.float32),
    )(ue_t, ie_t, w1a_t, w1b_t, b1c, w2c, b2c)


def kernel(u, i, user_emb, item_emb, W1, b1, W2, b2):
    u = u.astype(jnp.int32)
    i = i.astype(jnp.int32)
    uf, if_ = _tc_repack(user_emb.T, item_emb.T)
    ue_t, ie_t = _sc_gather(u, i, uf, if_)
    w1a_t = W1[:EMB].T
    w1b_t = W1[EMB:].T
    b1c = b1.reshape(EMB, 1)
    w2c = W2.reshape(EMB, 1)
    b2c = b2.reshape(1, 1)
    out_t = _tc_mlp(ue_t, ie_t, w1a_t, w1b_t, b1c, w2c, b2c)
    return out_t.reshape(BATCH, 1)


# speedup vs baseline: 19.9010x; 1.0119x over previous
import functools

import jax
import jax.numpy as jnp
from jax import lax
from jax.experimental import pallas as pl
from jax.experimental.pallas import tpu as pltpu
from jax.experimental.pallas import tpu_sc as plsc

EMB = 16
BATCH = 16384
TBL = 1000000
NC = 2
NS = 16
NW = NC * NS
BPW = BATCH // NW
CH = 128
L = 16
NG = BPW // L

TCOL = 131072
NBLK = (TBL + TCOL - 1) // TCOL
BLKW = 8 * TCOL
PLANE = NBLK * BLKW


def _repack_body(t_ref, out_ref):
    x = t_ref[...]
    out_ref[...] = x.reshape(8, TCOL // 128, 128).transpose(1, 0, 2).reshape(BLKW)


def _tc_repack(t):
    grid = (2, NBLK)
    return pl.pallas_call(
        _repack_body,
        grid=grid,
        in_specs=[pl.BlockSpec((8, TCOL), lambda p, m: (p, m))],
        out_specs=pl.BlockSpec((BLKW,), lambda p, m: (p * NBLK + m,)),
        out_shape=jax.ShapeDtypeStruct((2 * PLANE,), jnp.float32),
    )(t)


def _gather_body(idx_hbm, tab_hbm, out_hbm, idx_v, idxf, rows, sem):
    wid = lax.axis_index("s") * NC + lax.axis_index("c")
    base = wid * BPW
    pltpu.sync_copy(idx_hbm.at[pl.ds(base, BPW)], idx_v)
    for g in range(NG):
        r = idx_v[pl.ds(g * L, L)]
        t = ((r >> 7) << 10) + (r & 127)
        for c in range(EMB):
            off = (c // 8) * PLANE + (c % 8) * 128
            idxf[c, pl.ds(g * L, L)] = t + off
    copies = []
    for c in range(EMB):
        for ch in range(BPW // CH):
            off = ch * CH
            copies.append(pltpu.async_copy(
                tab_hbm.at[idxf.at[c, pl.ds(off, CH)]],
                rows.at[c, pl.ds(off, CH)], sem))
    for cp in copies:
        cp.wait()
    pltpu.sync_copy(rows, out_hbm.at[:, pl.ds(base, BPW)])


def _sc_gather(idx, flat):
    mesh = plsc.VectorSubcoreMesh(core_axis_name="c", subcore_axis_name="s")
    f = functools.partial(
        pl.kernel,
        mesh=mesh,
        out_type=jax.ShapeDtypeStruct((EMB, BATCH), jnp.float32),
        scratch_types=[
            pltpu.VMEM((BPW,), jnp.int32),
            pltpu.VMEM((EMB, BPW), jnp.int32),
            pltpu.VMEM((EMB, BPW), jnp.float32),
            pltpu.SemaphoreType.DMA,
        ],
        compiler_params=pltpu.CompilerParams(use_tc_tiling_on_sc=False),
    )(_gather_body)
    return f(idx, flat)


def _mlp_body(ue_ref, ie_ref, w1a_ref, w1b_ref, b1_ref, w2_ref, b2_ref, out_ref):
    h = jnp.dot(w1a_ref[...], ue_ref[...], preferred_element_type=jnp.float32)
    h = h + jnp.dot(w1b_ref[...], ie_ref[...], preferred_element_type=jnp.float32)
    h = jnp.maximum(h + b1_ref[...], 0.0)
    o = jnp.sum(h * w2_ref[...], axis=0, keepdims=True) + b2_ref[...]
    out_ref[...] = 1.0 / (1.0 + jnp.exp(-o))


BN = 4096


def _tc_mlp(ue_t, ie_t, w1a_t, w1b_t, b1c, w2c, b2c):
    grid = (BATCH // BN,)
    return pl.pallas_call(
        _mlp_body,
        grid=grid,
        in_specs=[
            pl.BlockSpec((EMB, BN), lambda m: (0, m)),
            pl.BlockSpec((EMB, BN), lambda m: (0, m)),
            pl.BlockSpec((EMB, EMB), lambda m: (0, 0)),
            pl.BlockSpec((EMB, EMB), lambda m: (0, 0)),
            pl.BlockSpec((EMB, 1), lambda m: (0, 0)),
            pl.BlockSpec((EMB, 1), lambda m: (0, 0)),
            pl.BlockSpec((1, 1), lambda m: (0, 0)),
        ],
        out_specs=pl.BlockSpec((1, BN), lambda m: (0, m)),
        out_shape=jax.ShapeDtypeStruct((1, BATCH), jnp.float32),
    )(ue_t, ie_t, w1a_t, w1b_t, b1c, w2c, b2c)


def kernel(u, i, user_emb, item_emb, W1, b1, W2, b2):
    u = u.astype(jnp.int32)
    i = i.astype(jnp.int32)
    uf = _tc_repack(user_emb.T)
    ue_t = _sc_gather(u, uf)
    if_ = _tc_repack(item_emb.T)
    ie_t = _sc_gather(i, if_)
    w1a_t = W1[:EMB].T
    w1b_t = W1[EMB:].T
    b1c = b1.reshape(EMB, 1)
    w2c = W2.reshape(EMB, 1)
    b2c = b2.reshape(1, 1)
    out_t = _tc_mlp(ue_t, ie_t, w1a_t, w1b_t, b1c, w2c, b2c)
    return out_t.reshape(BATCH, 1)


# speedup vs baseline: 20.3258x; 1.0213x over previous
import functools

import jax
import jax.numpy as jnp
from jax import lax
from jax.experimental import pallas as pl
from jax.experimental.pallas import tpu as pltpu
from jax.experimental.pallas import tpu_sc as plsc

EMB = 16
BATCH = 16384
TBL = 1000000
NC = 2
NS = 16
NW = NC * NS
BPW = BATCH // NW
CH = 128
L = 16
NG = BPW // L

TCOL = 262144
NBLK = (TBL + TCOL - 1) // TCOL
BLKW = 8 * TCOL
PLANE = NBLK * BLKW


def _repack_body(t_ref, out_ref):
    x = t_ref[...]
    out_ref[...] = x.reshape(8, TCOL // 128, 128).transpose(1, 0, 2).reshape(BLKW)


def _tc_repack(t):
    grid = (2, NBLK)
    return pl.pallas_call(
        _repack_body,
        grid=grid,
        in_specs=[pl.BlockSpec((8, TCOL), lambda p, m: (p, m))],
        out_specs=pl.BlockSpec((BLKW,), lambda p, m: (p * NBLK + m,)),
        out_shape=jax.ShapeDtypeStruct((2 * PLANE,), jnp.float32),
    )(t)


def _gather_body(idx_hbm, tab_hbm, out_hbm, idx_v, idxf, rows, sem):
    wid = lax.axis_index("s") * NC + lax.axis_index("c")
    base = wid * BPW
    pltpu.sync_copy(idx_hbm.at[pl.ds(base, BPW)], idx_v)
    for g in range(NG):
        r = idx_v[pl.ds(g * L, L)]
        t = ((r >> 7) << 10) + (r & 127)
        for c in range(EMB):
            off = (c // 8) * PLANE + (c % 8) * 128
            idxf[c, pl.ds(g * L, L)] = t + off
    copies = []
    for c in range(EMB):
        for ch in range(BPW // CH):
            off = ch * CH
            copies.append(pltpu.async_copy(
                tab_hbm.at[idxf.at[c, pl.ds(off, CH)]],
                rows.at[c, pl.ds(off, CH)], sem))
    for cp in copies:
        cp.wait()
    pltpu.sync_copy(rows, out_hbm.at[:, pl.ds(base, BPW)])


def _sc_gather(idx, flat):
    mesh = plsc.VectorSubcoreMesh(core_axis_name="c", subcore_axis_name="s")
    f = functools.partial(
        pl.kernel,
        mesh=mesh,
        out_type=jax.ShapeDtypeStruct((EMB, BATCH), jnp.float32),
        scratch_types=[
            pltpu.VMEM((BPW,), jnp.int32),
            pltpu.VMEM((EMB, BPW), jnp.int32),
            pltpu.VMEM((EMB, BPW), jnp.float32),
            pltpu.SemaphoreType.DMA,
        ],
        compiler_params=pltpu.CompilerParams(use_tc_tiling_on_sc=False),
    )(_gather_body)
    return f(idx, flat)


def _mlp_body(ue_ref, ie_ref, w1a_ref, w1b_ref, b1_ref, w2_ref, b2_ref, out_ref):
    h = jnp.dot(w1a_ref[...], ue_ref[...], preferred_element_type=jnp.float32)
    h = h + jnp.dot(w1b_ref[...], ie_ref[...], preferred_element_type=jnp.float32)
    h = jnp.maximum(h + b1_ref[...], 0.0)
    o = jnp.sum(h * w2_ref[...], axis=0, keepdims=True) + b2_ref[...]
    out_ref[...] = 1.0 / (1.0 + jnp.exp(-o))


BN = 4096


def _tc_mlp(ue_t, ie_t, w1a_t, w1b_t, b1c, w2c, b2c):
    grid = (BATCH // BN,)
    return pl.pallas_call(
        _mlp_body,
        grid=grid,
        in_specs=[
            pl.BlockSpec((EMB, BN), lambda m: (0, m)),
            pl.BlockSpec((EMB, BN), lambda m: (0, m)),
            pl.BlockSpec((EMB, EMB), lambda m: (0, 0)),
            pl.BlockSpec((EMB, EMB), lambda m: (0, 0)),
            pl.BlockSpec((EMB, 1), lambda m: (0, 0)),
            pl.BlockSpec((EMB, 1), lambda m: (0, 0)),
            pl.BlockSpec((1, 1), lambda m: (0, 0)),
        ],
        out_specs=pl.BlockSpec((1, BN), lambda m: (0, m)),
        out_shape=jax.ShapeDtypeStruct((1, BATCH), jnp.float32),
    )(ue_t, ie_t, w1a_t, w1b_t, b1c, w2c, b2c)


def kernel(u, i, user_emb, item_emb, W1, b1, W2, b2):
    u = u.astype(jnp.int32)
    i = i.astype(jnp.int32)
    uf = _tc_repack(user_emb.T)
    ue_t = _sc_gather(u, uf)
    if_ = _tc_repack(item_emb.T)
    ie_t = _sc_gather(i, if_)
    w1a_t = W1[:EMB].T
    w1b_t = W1[EMB:].T
    b1c = b1.reshape(EMB, 1)
    w2c = W2.reshape(EMB, 1)
    b2c = b2.reshape(1, 1)
    out_t = _tc_mlp(ue_t, ie_t, w1a_t, w1b_t, b1c, w2c, b2c)
    return out_t.reshape(BATCH, 1)
